# baseline (speedup 1.0000x reference)
import functools

import jax
import jax.numpy as jnp
import numpy as np
from jax.experimental import pallas as pl
from jax.experimental.pallas import tpu as pltpu

_NUM_CLASSES = 80
_C = 5 + _NUM_CLASSES
_N = 20
_ANCHORS = (
    np.array(
        [[10, 13], [16, 30], [33, 23], [30, 61], [62, 45], [59, 119],
         [116, 90], [156, 198], [373, 326]],
        dtype=np.float32,
    )
    / 608.0
)
_GRIDS = (19, 38, 76)
_LAMBDA_NOOBJ = 0.5
_IGNORE_THRESHOLD = 0.5
_BOX_GAIN = 0.05
_OBJ_GAIN = 1.0
_CLS_GAIN = 0.5
_EPS = 1e-9


def _softplus(x):
    return jnp.logaddexp(0.0, x)


def _prep_kernel(boxes_ref, labels_ref,
                 keys0_ref, meta0_ref, tcls0_ref, gidx0_ref,
                 keys1_ref, meta1_ref, tcls1_ref, gidx1_ref,
                 keys2_ref, meta2_ref, tcls2_ref, gidx2_ref):
    bx = boxes_ref[...]
    B = bx.shape[0]
    cx = bx[:, :, 0]
    cy = bx[:, :, 1]
    w = bx[:, :, 2]
    h = bx[:, :, 3]
    labels = labels_ref[...]

    idx9 = jax.lax.broadcasted_iota(jnp.int32, (1, 1, 9), 2)

    def _const_vec(vals):
        v = jnp.full((1, 1, 9), float(vals[8]), jnp.float32)
        for k in range(7, -1, -1):
            v = jnp.where(idx9 == k, float(vals[k]), v)
        return v

    aw9 = _const_vec(_ANCHORS[:, 0])
    ah9 = _const_vec(_ANCHORS[:, 1])
    inter = jnp.minimum(w[:, :, None], aw9) * jnp.minimum(h[:, :, None], ah9)
    union = (w * h)[:, :, None] + aw9 * ah9 - inter
    aiou = inter / (union + _EPS)
    mx = jnp.max(aiou, axis=2, keepdims=True)
    i9 = jax.lax.broadcasted_iota(jnp.int32, (B, _N, 9), 2)
    best = jnp.min(jnp.where(aiou == mx, i9, 9), axis=2)
    a3 = best % 3
    sidx = best // 3
    a3f = a3.astype(jnp.float32)

    lane85 = jax.lax.broadcasted_iota(jnp.int32, (B, _N, _C), 2)
    onehot = (lane85 == labels[:, :, None] + 5).astype(jnp.float32)

    jj = jax.lax.broadcasted_iota(jnp.int32, (B, _N, _N), 2)
    ii = jax.lax.broadcasted_iota(jnp.int32, (B, _N, _N), 1)
    jlt = jj < ii
    k16 = jax.lax.broadcasted_iota(jnp.int32, (B, _N, 16), 2)

    outs = ((keys0_ref, meta0_ref, tcls0_ref, gidx0_ref),
            (keys1_ref, meta1_ref, tcls1_ref, gidx1_ref),
            (keys2_ref, meta2_ref, tcls2_ref, gidx2_ref))
    for s in range(3):
        G = _GRIDS[s]
        keys_ref, meta_ref, tcls_ref, gidx_ref = outs[s]
        asg = sidx == s
        ix = jnp.clip(jnp.floor(cx * G), 0, G - 1).astype(jnp.int32)
        iy = jnp.clip(jnp.floor(cy * G), 0, G - 1).astype(jnp.int32)
        key = (a3 * G + ix) * G + iy
        keys_ref[...] = key.reshape(B, 1, _N)
        bofs = jax.lax.broadcasted_iota(jnp.int32, (B, _N), 0) * (3 * G * G)
        padrow = (jax.lax.broadcasted_iota(jnp.int32, (B, 12), 0) * 12
                  + jax.lax.broadcasted_iota(jnp.int32, (B, 12), 1))
        gidx = jnp.concatenate([key + bofs, padrow], axis=1)
        c85 = jax.lax.broadcasted_iota(jnp.int32, (B, 32, _C), 2)
        gidx_ref[...] = gidx[:, :, None] * _C + c85
        m3 = ((key[:, :, None] == key[:, None, :])
              & asg[:, :, None] & asg[:, None, :])
        anyprev = jnp.any(m3 & jlt, axis=2)
        leader = asg & jnp.logical_not(anyprev)
        lf = leader.astype(jnp.float32)
        mf = m3.astype(jnp.float32)
        sx = jnp.sum(mf * cx[:, None, :], axis=2)
        sy = jnp.sum(mf * cy[:, None, :], axis=2)
        sw = jnp.sum(mf * w[:, None, :], axis=2)
        sh = jnp.sum(mf * h[:, None, :], axis=2)
        tcls = jnp.sum(mf[:, :, :, None] * onehot[:, None, :, :], axis=2)
        tcls_ref[...] = tcls

        def col(v):
            return v[:, :, None]

        meta = jnp.where(
            k16 == 0, col(lf),
            jnp.where(k16 == 1, col(a3f),
                      jnp.where(k16 == 2, col(ix.astype(jnp.float32)),
                                jnp.where(k16 == 3, col(iy.astype(jnp.float32)),
                                          jnp.where(k16 == 4, col(sx),
                                                    jnp.where(k16 == 5, col(sy),
                                                              jnp.where(k16 == 6, col(sw),
                                                                        col(sh))))))))
        meta_ref[...] = meta


def _scale_kernel(s, G, R, planes_ref, boxes_s_ref, boxesT_ref,
                  meta_ref, tcls_ref, rows_ref,
                  loc_ref, cls_ref, objc_ref, noobj_ref, cnt_ref):
    b = pl.program_id(0)
    GG = G * G
    GG3 = 3 * GG

    @pl.when(b == 0)
    def _init():
        loc_ref[0] = 0.0
        cls_ref[0] = 0.0
        objc_ref[0] = 0.0
        noobj_ref[0] = 0.0
        cnt_ref[0] = 0.0

    q = (jax.lax.broadcasted_iota(jnp.int32, (R, 128), 0) * 128
         + jax.lax.broadcasted_iota(jnp.int32, (R, 128), 1))
    a_idx = q // GG
    rem_q = q - a_idx * GG
    gx = (rem_q // G).astype(jnp.float32)
    gy = (rem_q - (rem_q // G) * G).astype(jnp.float32)
    valid = q < GG3
    aw0, aw1, aw2 = (float(_ANCHORS[3 * s + k, 0]) for k in range(3))
    ah0, ah1, ah2 = (float(_ANCHORS[3 * s + k, 1]) for k in range(3))
    awv = jnp.where(a_idx == 0, aw0, jnp.where(a_idx == 1, aw1, aw2))
    ahv = jnp.where(a_idx == 0, ah0, jnp.where(a_idx == 1, ah1, ah2))

    x0 = planes_ref[0, 0]
    x1 = planes_ref[0, 1]
    x2 = planes_ref[0, 2]
    x3 = planes_ref[0, 3]
    x4 = planes_ref[0, 4]
    px = (jax.nn.sigmoid(x0) + gx) * (1.0 / G)
    py = (jax.nn.sigmoid(x1) + gy) * (1.0 / G)
    pw = jnp.exp(x2) * awv
    ph = jnp.exp(x3) * ahv
    p1x = px - pw * 0.5
    p2x = px + pw * 0.5
    p1y = py - ph * 0.5
    p2y = py + ph * 0.5
    area_p = pw * ph
    ign = jnp.zeros((R, 128), jnp.bool_)
    for n in range(_N):
        bcx = boxes_s_ref[0, 0, 4 * n + 0]
        bcy = boxes_s_ref[0, 0, 4 * n + 1]
        bw = boxes_s_ref[0, 0, 4 * n + 2]
        bh = boxes_s_ref[0, 0, 4 * n + 3]
        iw = jnp.maximum(
            jnp.minimum(p2x, bcx + bw * 0.5) - jnp.maximum(p1x, bcx - bw * 0.5), 0.0)
        ih = jnp.maximum(
            jnp.minimum(p2y, bcy + bh * 0.5) - jnp.maximum(p1y, bcy - bh * 0.5), 0.0)
        inter = iw * ih
        ign = ign | (3.0 * inter > area_p + bw * bh)
    noobj_acc = jnp.sum(jnp.where(ign | (~valid), 0.0, _softplus(x4)))

    meta = meta_ref[0]
    lf = meta[:, 0:1]
    af = meta[:, 1:2]
    ixf = meta[:, 2:3]
    iyf = meta[:, 3:4]
    sx = meta[:, 4:5]
    sy = meta[:, 5:6]
    sw = meta[:, 6:7]
    sh = meta[:, 7:8]
    rows = rows_ref[0, 0:_N, :]
    tcls = tcls_ref[0]
    lane = jax.lax.broadcasted_iota(jnp.int32, (1, _C), 1)

    sig = jax.nn.sigmoid(rows)
    ex = jnp.exp(rows)
    sp = _softplus(rows)
    aw0f, aw1f, aw2f = (float(_ANCHORS[3 * s + k, 0]) for k in range(3))
    ah0f, ah1f, ah2f = (float(_ANCHORS[3 * s + k, 1]) for k in range(3))
    awcol = jnp.where(af == 0.0, aw0f, jnp.where(af == 1.0, aw1f, aw2f))
    ahcol = jnp.where(af == 0.0, ah0f, jnp.where(af == 1.0, ah1f, ah2f))
    addv = jnp.where(lane == 0, ixf, iyf)
    anchv = jnp.where(lane == 2, awcol, ahcol)
    tp_xy = (sig + addv) * (1.0 / G)
    tp_wh = ex * anchv
    tgt = jnp.where(lane == 0, sx,
                    jnp.where(lane == 1, sy,
                              jnp.where(lane == 2, sw, sh)))
    m_xy = (lane <= 1).astype(jnp.float32)
    m_wh = ((lane == 2) | (lane == 3)).astype(jnp.float32)
    m_obj = (lane == 4).astype(jnp.float32)
    m_cls = (lane >= 5).astype(jnp.float32)
    dxy = tp_xy - tgt
    dwh = tp_wh - tgt
    loc_c = jnp.sum(lf * (dxy * dxy * m_xy + dwh * dwh * m_wh))
    cls_c = jnp.sum(lf * (sp - rows * tcls) * m_cls)
    po = jnp.sum(rows * m_obj, axis=1, keepdims=True)
    posp = jnp.sum(sp * m_obj, axis=1, keepdims=True)
    pxc = jnp.sum(tp_xy * (lane == 0), axis=1, keepdims=True)
    pyc = jnp.sum(tp_xy * (lane == 1), axis=1, keepdims=True)
    pwc = jnp.sum(tp_wh * (lane == 2), axis=1, keepdims=True)
    phc = jnp.sum(tp_wh * (lane == 3), axis=1, keepdims=True)

    bt = boxesT_ref[0]
    gcxr = bt[0:1, :]
    gcyr = bt[1:2, :]
    gwr = bt[2:3, :]
    ghr = bt[3:4, :]
    g1xr = gcxr - gwr * 0.5
    g2xr = gcxr + gwr * 0.5
    g1yr = gcyr - ghr * 0.5
    g2yr = gcyr + ghr * 0.5
    area_gr = gwr * ghr
    iw2 = jnp.maximum(
        jnp.minimum(pxc + pwc * 0.5, g2xr) - jnp.maximum(pxc - pwc * 0.5, g1xr), 0.0)
    ih2 = jnp.maximum(
        jnp.minimum(pyc + phc * 0.5, g2yr) - jnp.maximum(pyc - phc * 0.5, g1yr), 0.0)
    inter2 = iw2 * ih2
    iou2 = inter2 / (pwc * phc + area_gr - inter2 + _EPS)
    ignf = (jnp.max(iou2, axis=1, keepdims=True) > _IGNORE_THRESHOLD)
    objc_c = jnp.sum(lf * ((posp - po)
                           - jnp.where(ignf, 0.0, 0.5 * posp)))
    cnt_c = jnp.sum(lf)

    loc_ref[0] += loc_c
    cls_ref[0] += cls_c
    objc_ref[0] += objc_c
    noobj_ref[0] += noobj_acc
    cnt_ref[0] += cnt_c


def _run_prep(boxes, labels_i):
    B = boxes.shape[0]
    outs = pl.pallas_call(
        _prep_kernel,
        in_specs=[
            pl.BlockSpec((B, _N, 4), lambda: (0, 0, 0)),
            pl.BlockSpec((B, _N), lambda: (0, 0)),
        ],
        out_specs=[
            spec
            for _ in range(3)
            for spec in (
                pl.BlockSpec((B, 1, _N), lambda: (0, 0, 0)),
                pl.BlockSpec((B, _N, 16), lambda: (0, 0, 0)),
                pl.BlockSpec((B, _N, _C), lambda: (0, 0, 0)),
                pl.BlockSpec((B, 32, _C), lambda: (0, 0, 0)),
            )
        ],
        out_shape=[
            shape
            for _ in range(3)
            for shape in (
                jax.ShapeDtypeStruct((B, 1, _N), jnp.int32),
                jax.ShapeDtypeStruct((B, _N, 16), jnp.float32),
                jax.ShapeDtypeStruct((B, _N, _C), jnp.float32),
                jax.ShapeDtypeStruct((B, 32, _C), jnp.int32),
            )
        ],
    )(boxes, labels_i)
    return outs


def _sc_gather(preds, gidxs):
    import jax.experimental.pallas.tpu_sc as plsc
    from jax import lax

    info = plsc.get_sparse_core_info()
    NC, NS = info.num_cores, info.num_subcores
    NW = NC * NS
    B = gidxs[0].shape[0]
    mesh = plsc.VectorSubcoreMesh(core_axis_name="c", subcore_axis_name="s")

    @functools.partial(
        pl.kernel, mesh=mesh,
        out_type=[jax.ShapeDtypeStruct((NW * 1408,), jnp.float32)
                  for _ in range(3)],
        scratch_types=[
            pltpu.VMEM((1408,), jnp.int32),
            pltpu.VMEM((1408,), jnp.float32),
            pltpu.SemaphoreType.DMA,
        ],
        compiler_params=pltpu.CompilerParams(use_tc_tiling_on_sc=False),
    )
    def gk(tab0, tab1, tab2, idx0, idx1, idx2, out0, out1, out2,
           idx_v, rows_v, sem):
        wid = lax.axis_index("s") * NC + lax.axis_index("c")
        base = wid * 1408
        for tab, idx, out in ((tab0, idx0, out0), (tab1, idx1, out1),
                              (tab2, idx2, out2)):
            pltpu.sync_copy(idx.at[pl.ds(base, 1408)], idx_v)
            pltpu.async_copy(tab.at[idx_v], rows_v, sem).wait()
            pltpu.sync_copy(rows_v, out.at[pl.ds(base, 1408)])

    tabs = [p.reshape(p.size) for p in preds]
    padfill = (jnp.arange(NW, dtype=jnp.int32)[:, None] * 48
               + jnp.arange(48, dtype=jnp.int32)[None, :])
    flat_idx = []
    for g in gidxs:
        g = g.reshape(NW, 1360)
        g = jnp.concatenate([g, padfill], axis=1)
        flat_idx.append(g.reshape(NW * 1408))
    rows = gk(tabs[0], tabs[1], tabs[2],
              flat_idx[0], flat_idx[1], flat_idx[2])
    out = []
    for r in rows:
        r = r.reshape(NW, 1408)[:, :1360]
        out.append(r.reshape(B, 32, _C))
    return out


def _run_scale(s, planes, boxes_flat, boxesT, meta, tcls, rows):
    B = planes.shape[0]
    G = _GRIDS[s]
    R = planes.shape[2]
    outs = pl.pallas_call(
        functools.partial(_scale_kernel, s, G, R),
        grid=(B,),
        in_specs=[
            pl.BlockSpec((1, 5, R, 128), lambda b: (b, 0, 0, 0)),
            pl.BlockSpec((1, 1, 4 * _N), lambda b: (b, 0, 0), memory_space=pltpu.SMEM),
            pl.BlockSpec((1, 4, _N), lambda b: (b, 0, 0)),
            pl.BlockSpec((1, _N, 16), lambda b: (b, 0, 0)),
            pl.BlockSpec((1, _N, _C), lambda b: (b, 0, 0)),
            pl.BlockSpec((1, 32, _C), lambda b: (b, 0, 0)),
        ],
        out_specs=[
            pl.BlockSpec(memory_space=pltpu.SMEM) for _ in range(5)
        ],
        out_shape=[jax.ShapeDtypeStruct((1,), jnp.float32) for _ in range(5)],
        compiler_params=pltpu.CompilerParams(
            dimension_semantics=("arbitrary",)),
    )(planes, boxes_flat, boxesT, meta, tcls, rows)
    return outs


def _make_planes(pred):
    B = pred.shape[0]
    G = pred.shape[2]
    GG3 = 3 * G * G
    p5 = pred.reshape(B, GG3, _C)[:, :, 0:5]
    p5 = jax.lax.optimization_barrier(p5)
    planes = jnp.moveaxis(p5, 2, 1)
    R = (GG3 + 127) // 128
    pad = R * 128 - GG3
    planes = jnp.pad(planes, ((0, 0), (0, 0), (0, pad)))
    return planes.reshape(B, 5, R, 128)


def kernel(pred_large, pred_medium, pred_small, boxes, labels):
    B = pred_large.shape[0]
    boxes_flat = boxes.reshape(B, 1, 4 * _N)
    boxesT = jnp.swapaxes(boxes, 1, 2)
    labels_i = labels.astype(jnp.int32)
    prep = _run_prep(boxes, labels_i)
    preds = [pred_large, pred_medium, pred_small]
    gidxs = [prep[4 * s + 3] for s in range(3)]
    rows_all = _sc_gather(preds, gidxs)
    loc = jnp.float32(0.0)
    cls = jnp.float32(0.0)
    obj = jnp.float32(0.0)
    cnt = jnp.float32(0.0)
    for s, pred in enumerate(preds):
        planes = _make_planes(pred)
        meta, tcls = prep[4 * s + 1], prep[4 * s + 2]
        o_loc, o_cls, o_objc, o_noobj, o_cnt = _run_scale(
            s, planes, boxes_flat, boxesT, meta, tcls, rows_all[s])
        loc = loc + o_loc[0]
        cls = cls + o_cls[0]
        obj = obj + o_objc[0] + _LAMBDA_NOOBJ * o_noobj[0]
        cnt = cnt + o_cnt[0]
    denom = jnp.maximum(1.0, cnt)
    loc_loss = loc / denom
    cls_loss = cls / denom
    obj_loss = obj / B
    total_loss = _BOX_GAIN * loc_loss + _OBJ_GAIN * obj_loss + _CLS_GAIN * cls_loss
    return total_loss, loc_loss, obj_loss, cls_loss


# speedup vs baseline: 1.6250x; 1.6250x over previous
import functools

import jax
import jax.numpy as jnp
import numpy as np
from jax.experimental import pallas as pl
from jax.experimental.pallas import tpu as pltpu

_NUM_CLASSES = 80
_C = 5 + _NUM_CLASSES
_N = 20
_ANCHORS = (
    np.array(
        [[10, 13], [16, 30], [33, 23], [30, 61], [62, 45], [59, 119],
         [116, 90], [156, 198], [373, 326]],
        dtype=np.float32,
    )
    / 608.0
)
_GRIDS = (19, 38, 76)
_LAMBDA_NOOBJ = 0.5
_IGNORE_THRESHOLD = 0.5
_BOX_GAIN = 0.05
_OBJ_GAIN = 1.0
_CLS_GAIN = 0.5
_EPS = 1e-9


def _softplus(x):
    return jnp.logaddexp(0.0, x)


def _prep_kernel(boxes_ref, labels_ref,
                 keys0_ref, meta0_ref, tcls0_ref,
                 keys1_ref, meta1_ref, tcls1_ref,
                 keys2_ref, meta2_ref, tcls2_ref):
    bx = boxes_ref[...]
    B = bx.shape[0]
    cx = bx[:, :, 0]
    cy = bx[:, :, 1]
    w = bx[:, :, 2]
    h = bx[:, :, 3]
    labels = labels_ref[...]

    idx9 = jax.lax.broadcasted_iota(jnp.int32, (1, 1, 9), 2)

    def _const_vec(vals):
        v = jnp.full((1, 1, 9), float(vals[8]), jnp.float32)
        for k in range(7, -1, -1):
            v = jnp.where(idx9 == k, float(vals[k]), v)
        return v

    aw9 = _const_vec(_ANCHORS[:, 0])
    ah9 = _const_vec(_ANCHORS[:, 1])
    inter = jnp.minimum(w[:, :, None], aw9) * jnp.minimum(h[:, :, None], ah9)
    union = (w * h)[:, :, None] + aw9 * ah9 - inter
    aiou = inter / (union + _EPS)
    mx = jnp.max(aiou, axis=2, keepdims=True)
    i9 = jax.lax.broadcasted_iota(jnp.int32, (B, _N, 9), 2)
    best = jnp.min(jnp.where(aiou == mx, i9, 9), axis=2)
    a3 = best % 3
    sidx = best // 3
    a3f = a3.astype(jnp.float32)

    lane85 = jax.lax.broadcasted_iota(jnp.int32, (B, _N, _C), 2)
    onehot = (lane85 == labels[:, :, None] + 5).astype(jnp.float32)

    jj = jax.lax.broadcasted_iota(jnp.int32, (B, _N, _N), 2)
    ii = jax.lax.broadcasted_iota(jnp.int32, (B, _N, _N), 1)
    jlt = jj < ii
    k16 = jax.lax.broadcasted_iota(jnp.int32, (B, _N, 16), 2)

    outs = ((keys0_ref, meta0_ref, tcls0_ref),
            (keys1_ref, meta1_ref, tcls1_ref),
            (keys2_ref, meta2_ref, tcls2_ref))
    for s in range(3):
        G = _GRIDS[s]
        keys_ref, meta_ref, tcls_ref = outs[s]
        asg = sidx == s
        ix = jnp.clip(jnp.floor(cx * G), 0, G - 1).astype(jnp.int32)
        iy = jnp.clip(jnp.floor(cy * G), 0, G - 1).astype(jnp.int32)
        key = (a3 * G + ix) * G + iy
        keys_ref[...] = key.reshape(B, 1, _N)
        m3 = ((key[:, :, None] == key[:, None, :])
              & asg[:, :, None] & asg[:, None, :])
        anyprev = jnp.any(m3 & jlt, axis=2)
        leader = asg & jnp.logical_not(anyprev)
        lf = leader.astype(jnp.float32)
        mf = m3.astype(jnp.float32)
        sx = jnp.sum(mf * cx[:, None, :], axis=2)
        sy = jnp.sum(mf * cy[:, None, :], axis=2)
        sw = jnp.sum(mf * w[:, None, :], axis=2)
        sh = jnp.sum(mf * h[:, None, :], axis=2)
        tcls = jnp.sum(mf[:, :, :, None] * onehot[:, None, :, :], axis=2)
        tcls_ref[...] = tcls

        def col(v):
            return v[:, :, None]

        meta = jnp.where(
            k16 == 0, col(lf),
            jnp.where(k16 == 1, col(a3f),
                      jnp.where(k16 == 2, col(ix.astype(jnp.float32)),
                                jnp.where(k16 == 3, col(iy.astype(jnp.float32)),
                                          jnp.where(k16 == 4, col(sx),
                                                    jnp.where(k16 == 5, col(sy),
                                                              jnp.where(k16 == 6, col(sw),
                                                                        col(sh))))))))
        meta_ref[...] = meta


def _scale_kernel(s, G, R, pred_hbm, planes_ref, boxes_s_ref, boxesT_ref,
                  keys_ref, meta_ref, tcls_ref,
                  loc_ref, cls_ref, objc_ref, noobj_ref, cnt_ref,
                  rows_sc, sem):
    b = pl.program_id(0)
    GG = G * G
    GG3 = 3 * GG

    @pl.when(b == 0)
    def _init():
        loc_ref[0] = 0.0
        cls_ref[0] = 0.0
        objc_ref[0] = 0.0
        noobj_ref[0] = 0.0
        cnt_ref[0] = 0.0

    for i in range(_N):
        key = keys_ref[0, 0, i]
        a = key // GG
        rem = key - a * GG
        ix = rem // G
        iy = rem - ix * G
        pltpu.make_async_copy(
            pred_hbm.at[b, a, ix, iy], rows_sc.at[i], sem).start()

    q = (jax.lax.broadcasted_iota(jnp.int32, (R, 128), 0) * 128
         + jax.lax.broadcasted_iota(jnp.int32, (R, 128), 1))
    a_idx = q // GG
    rem_q = q - a_idx * GG
    gx = (rem_q // G).astype(jnp.float32)
    gy = (rem_q - (rem_q // G) * G).astype(jnp.float32)
    valid = q < GG3
    aw0, aw1, aw2 = (float(_ANCHORS[3 * s + k, 0]) for k in range(3))
    ah0, ah1, ah2 = (float(_ANCHORS[3 * s + k, 1]) for k in range(3))
    awv = jnp.where(a_idx == 0, aw0, jnp.where(a_idx == 1, aw1, aw2))
    ahv = jnp.where(a_idx == 0, ah0, jnp.where(a_idx == 1, ah1, ah2))

    x0 = planes_ref[0, 0]
    x1 = planes_ref[0, 1]
    x2 = planes_ref[0, 2]
    x3 = planes_ref[0, 3]
    x4 = planes_ref[0, 4]
    px = (jax.nn.sigmoid(x0) + gx) * (1.0 / G)
    py = (jax.nn.sigmoid(x1) + gy) * (1.0 / G)
    pw = jnp.exp(x2) * awv
    ph = jnp.exp(x3) * ahv
    p1x = px - pw * 0.5
    p2x = px + pw * 0.5
    p1y = py - ph * 0.5
    p2y = py + ph * 0.5
    area_p = pw * ph
    ign = jnp.zeros((R, 128), jnp.bool_)
    for n in range(_N):
        bcx = boxes_s_ref[0, 0, 4 * n + 0]
        bcy = boxes_s_ref[0, 0, 4 * n + 1]
        bw = boxes_s_ref[0, 0, 4 * n + 2]
        bh = boxes_s_ref[0, 0, 4 * n + 3]
        iw = jnp.maximum(
            jnp.minimum(p2x, bcx + bw * 0.5) - jnp.maximum(p1x, bcx - bw * 0.5), 0.0)
        ih = jnp.maximum(
            jnp.minimum(p2y, bcy + bh * 0.5) - jnp.maximum(p1y, bcy - bh * 0.5), 0.0)
        inter = iw * ih
        ign = ign | (3.0 * inter > area_p + bw * bh)
    noobj_acc = jnp.sum(jnp.where(ign | (~valid), 0.0, _softplus(x4)))

    for i in range(_N):
        pltpu.make_async_copy(
            pred_hbm.at[b, 0, 0, 0], rows_sc.at[i], sem).wait()

    meta = meta_ref[0]
    lf = meta[:, 0:1]
    af = meta[:, 1:2]
    ixf = meta[:, 2:3]
    iyf = meta[:, 3:4]
    sx = meta[:, 4:5]
    sy = meta[:, 5:6]
    sw = meta[:, 6:7]
    sh = meta[:, 7:8]
    rows = rows_sc[:, :]
    tcls = tcls_ref[0]
    lane = jax.lax.broadcasted_iota(jnp.int32, (1, _C), 1)

    sig = jax.nn.sigmoid(rows)
    ex = jnp.exp(rows)
    sp = _softplus(rows)
    aw0f, aw1f, aw2f = (float(_ANCHORS[3 * s + k, 0]) for k in range(3))
    ah0f, ah1f, ah2f = (float(_ANCHORS[3 * s + k, 1]) for k in range(3))
    awcol = jnp.where(af == 0.0, aw0f, jnp.where(af == 1.0, aw1f, aw2f))
    ahcol = jnp.where(af == 0.0, ah0f, jnp.where(af == 1.0, ah1f, ah2f))
    addv = jnp.where(lane == 0, ixf, iyf)
    anchv = jnp.where(lane == 2, awcol, ahcol)
    tp_xy = (sig + addv) * (1.0 / G)
    tp_wh = ex * anchv
    tgt = jnp.where(lane == 0, sx,
                    jnp.where(lane == 1, sy,
                              jnp.where(lane == 2, sw, sh)))
    m_xy = (lane <= 1).astype(jnp.float32)
    m_wh = ((lane == 2) | (lane == 3)).astype(jnp.float32)
    m_obj = (lane == 4).astype(jnp.float32)
    m_cls = (lane >= 5).astype(jnp.float32)
    dxy = tp_xy - tgt
    dwh = tp_wh - tgt
    loc_c = jnp.sum(lf * (dxy * dxy * m_xy + dwh * dwh * m_wh))
    cls_c = jnp.sum(lf * (sp - rows * tcls) * m_cls)
    po = jnp.sum(rows * m_obj, axis=1, keepdims=True)
    posp = jnp.sum(sp * m_obj, axis=1, keepdims=True)
    pxc = jnp.sum(tp_xy * (lane == 0), axis=1, keepdims=True)
    pyc = jnp.sum(tp_xy * (lane == 1), axis=1, keepdims=True)
    pwc = jnp.sum(tp_wh * (lane == 2), axis=1, keepdims=True)
    phc = jnp.sum(tp_wh * (lane == 3), axis=1, keepdims=True)

    bt = boxesT_ref[0]
    gcxr = bt[0:1, :]
    gcyr = bt[1:2, :]
    gwr = bt[2:3, :]
    ghr = bt[3:4, :]
    g1xr = gcxr - gwr * 0.5
    g2xr = gcxr + gwr * 0.5
    g1yr = gcyr - ghr * 0.5
    g2yr = gcyr + ghr * 0.5
    area_gr = gwr * ghr
    iw2 = jnp.maximum(
        jnp.minimum(pxc + pwc * 0.5, g2xr) - jnp.maximum(pxc - pwc * 0.5, g1xr), 0.0)
    ih2 = jnp.maximum(
        jnp.minimum(pyc + phc * 0.5, g2yr) - jnp.maximum(pyc - phc * 0.5, g1yr), 0.0)
    inter2 = iw2 * ih2
    iou2 = inter2 / (pwc * phc + area_gr - inter2 + _EPS)
    ignf = (jnp.max(iou2, axis=1, keepdims=True) > _IGNORE_THRESHOLD)
    objc_c = jnp.sum(lf * ((posp - po)
                           - jnp.where(ignf, 0.0, 0.5 * posp)))
    cnt_c = jnp.sum(lf)

    loc_ref[0] += loc_c
    cls_ref[0] += cls_c
    objc_ref[0] += objc_c
    noobj_ref[0] += noobj_acc
    cnt_ref[0] += cnt_c


def _run_prep(boxes, labels_i):
    B = boxes.shape[0]
    outs = pl.pallas_call(
        _prep_kernel,
        in_specs=[
            pl.BlockSpec((B, _N, 4), lambda: (0, 0, 0)),
            pl.BlockSpec((B, _N), lambda: (0, 0)),
        ],
        out_specs=[
            spec
            for _ in range(3)
            for spec in (
                pl.BlockSpec((B, 1, _N), lambda: (0, 0, 0)),
                pl.BlockSpec((B, _N, 16), lambda: (0, 0, 0)),
                pl.BlockSpec((B, _N, _C), lambda: (0, 0, 0)),
            )
        ],
        out_shape=[
            shape
            for _ in range(3)
            for shape in (
                jax.ShapeDtypeStruct((B, 1, _N), jnp.int32),
                jax.ShapeDtypeStruct((B, _N, 16), jnp.float32),
                jax.ShapeDtypeStruct((B, _N, _C), jnp.float32),
            )
        ],
    )(boxes, labels_i)
    return outs


def _run_scale(s, pred, planes, boxes_flat, boxesT, keys, meta, tcls):
    B = pred.shape[0]
    G = pred.shape[2]
    R = planes.shape[2]
    outs = pl.pallas_call(
        functools.partial(_scale_kernel, s, G, R),
        grid=(B,),
        in_specs=[
            pl.BlockSpec(memory_space=pl.ANY),
            pl.BlockSpec((1, 5, R, 128), lambda b: (b, 0, 0, 0)),
            pl.BlockSpec((1, 1, 4 * _N), lambda b: (b, 0, 0), memory_space=pltpu.SMEM),
            pl.BlockSpec((1, 4, _N), lambda b: (b, 0, 0)),
            pl.BlockSpec((1, 1, _N), lambda b: (b, 0, 0), memory_space=pltpu.SMEM),
            pl.BlockSpec((1, _N, 16), lambda b: (b, 0, 0)),
            pl.BlockSpec((1, _N, _C), lambda b: (b, 0, 0)),
        ],
        out_specs=[
            pl.BlockSpec(memory_space=pltpu.SMEM) for _ in range(5)
        ],
        out_shape=[jax.ShapeDtypeStruct((1,), jnp.float32) for _ in range(5)],
        scratch_shapes=[
            pltpu.VMEM((_N, _C), jnp.float32),
            pltpu.SemaphoreType.DMA,
        ],
        compiler_params=pltpu.CompilerParams(
            dimension_semantics=("arbitrary",)),
    )(pred, planes, boxes_flat, boxesT, keys, meta, tcls)
    return outs


def _make_planes(pred):
    B = pred.shape[0]
    G = pred.shape[2]
    GG3 = 3 * G * G
    p5 = pred.reshape(B, GG3, _C)[:, :, 0:5]
    p5 = jax.lax.optimization_barrier(p5)
    planes = jnp.moveaxis(p5, 2, 1)
    R = (GG3 + 127) // 128
    pad = R * 128 - GG3
    planes = jnp.pad(planes, ((0, 0), (0, 0), (0, pad)))
    return planes.reshape(B, 5, R, 128)


def kernel(pred_large, pred_medium, pred_small, boxes, labels):
    B = pred_large.shape[0]
    boxes_flat = boxes.reshape(B, 1, 4 * _N)
    boxesT = jnp.swapaxes(boxes, 1, 2)
    labels_i = labels.astype(jnp.int32)
    prep = _run_prep(boxes, labels_i)
    loc = jnp.float32(0.0)
    cls = jnp.float32(0.0)
    obj = jnp.float32(0.0)
    cnt = jnp.float32(0.0)
    for s, pred in enumerate([pred_large, pred_medium, pred_small]):
        planes = _make_planes(pred)
        keys, meta, tcls = prep[3 * s], prep[3 * s + 1], prep[3 * s + 2]
        o_loc, o_cls, o_objc, o_noobj, o_cnt = _run_scale(
            s, pred, planes, boxes_flat, boxesT, keys, meta, tcls)
        loc = loc + o_loc[0]
        cls = cls + o_cls[0]
        obj = obj + o_objc[0] + _LAMBDA_NOOBJ * o_noobj[0]
        cnt = cnt + o_cnt[0]
    denom = jnp.maximum(1.0, cnt)
    loc_loss = loc / denom
    cls_loss = cls / denom
    obj_loss = obj / B
    total_loss = _BOX_GAIN * loc_loss + _OBJ_GAIN * obj_loss + _CLS_GAIN * cls_loss
    return total_loss, loc_loss, obj_loss, cls_loss


# speedup vs baseline: 1.6255x; 1.0003x over previous
import functools

import jax
import jax.numpy as jnp
import numpy as np
from jax.experimental import pallas as pl
from jax.experimental.pallas import tpu as pltpu

_NUM_CLASSES = 80
_C = 5 + _NUM_CLASSES
_N = 20
_ANCHORS = (
    np.array(
        [[10, 13], [16, 30], [33, 23], [30, 61], [62, 45], [59, 119],
         [116, 90], [156, 198], [373, 326]],
        dtype=np.float32,
    )
    / 608.0
)
_GRIDS = (19, 38, 76)
_LAMBDA_NOOBJ = 0.5
_IGNORE_THRESHOLD = 0.5
_BOX_GAIN = 0.05
_OBJ_GAIN = 1.0
_CLS_GAIN = 0.5
_EPS = 1e-9


def _softplus(x):
    return jnp.logaddexp(0.0, x)


def _prep_kernel(boxes_ref, labels_ref,
                 keys0_ref, meta0_ref, tcls0_ref,
                 keys1_ref, meta1_ref, tcls1_ref,
                 keys2_ref, meta2_ref, tcls2_ref):
    bx = boxes_ref[...]
    B = bx.shape[0]
    cx = bx[:, :, 0]
    cy = bx[:, :, 1]
    w = bx[:, :, 2]
    h = bx[:, :, 3]
    labels = labels_ref[...]

    idx9 = jax.lax.broadcasted_iota(jnp.int32, (1, 1, 9), 2)

    def _const_vec(vals):
        v = jnp.full((1, 1, 9), float(vals[8]), jnp.float32)
        for k in range(7, -1, -1):
            v = jnp.where(idx9 == k, float(vals[k]), v)
        return v

    aw9 = _const_vec(_ANCHORS[:, 0])
    ah9 = _const_vec(_ANCHORS[:, 1])
    inter = jnp.minimum(w[:, :, None], aw9) * jnp.minimum(h[:, :, None], ah9)
    union = (w * h)[:, :, None] + aw9 * ah9 - inter
    aiou = inter / (union + _EPS)
    mx = jnp.max(aiou, axis=2, keepdims=True)
    i9 = jax.lax.broadcasted_iota(jnp.int32, (B, _N, 9), 2)
    best = jnp.min(jnp.where(aiou == mx, i9, 9), axis=2)
    a3 = best % 3
    sidx = best // 3
    a3f = a3.astype(jnp.float32)

    lane85 = jax.lax.broadcasted_iota(jnp.int32, (B, _N, _C), 2)
    onehot = (lane85 == labels[:, :, None] + 5).astype(jnp.float32)

    jj = jax.lax.broadcasted_iota(jnp.int32, (B, _N, _N), 2)
    ii = jax.lax.broadcasted_iota(jnp.int32, (B, _N, _N), 1)
    jlt = jj < ii
    k16 = jax.lax.broadcasted_iota(jnp.int32, (B, _N, 16), 2)

    outs = ((keys0_ref, meta0_ref, tcls0_ref),
            (keys1_ref, meta1_ref, tcls1_ref),
            (keys2_ref, meta2_ref, tcls2_ref))
    for s in range(3):
        G = _GRIDS[s]
        keys_ref, meta_ref, tcls_ref = outs[s]
        asg = sidx == s
        ix = jnp.clip(jnp.floor(cx * G), 0, G - 1).astype(jnp.int32)
        iy = jnp.clip(jnp.floor(cy * G), 0, G - 1).astype(jnp.int32)
        key = (a3 * G + ix) * G + iy
        keys_ref[...] = key.reshape(B, 1, _N)
        m3 = ((key[:, :, None] == key[:, None, :])
              & asg[:, :, None] & asg[:, None, :])
        anyprev = jnp.any(m3 & jlt, axis=2)
        leader = asg & jnp.logical_not(anyprev)
        lf = leader.astype(jnp.float32)
        mf = m3.astype(jnp.float32)
        sx = jnp.sum(mf * cx[:, None, :], axis=2)
        sy = jnp.sum(mf * cy[:, None, :], axis=2)
        sw = jnp.sum(mf * w[:, None, :], axis=2)
        sh = jnp.sum(mf * h[:, None, :], axis=2)
        tcls = jnp.sum(mf[:, :, :, None] * onehot[:, None, :, :], axis=2)
        tcls_ref[...] = tcls

        def col(v):
            return v[:, :, None]

        meta = jnp.where(
            k16 == 0, col(lf),
            jnp.where(k16 == 1, col(a3f),
                      jnp.where(k16 == 2, col(ix.astype(jnp.float32)),
                                jnp.where(k16 == 3, col(iy.astype(jnp.float32)),
                                          jnp.where(k16 == 4, col(sx),
                                                    jnp.where(k16 == 5, col(sy),
                                                              jnp.where(k16 == 6, col(sw),
                                                                        col(sh))))))))
        meta_ref[...] = meta


def _scale_kernel(s, G, R, pred_hbm, planes_ref, boxes_s_ref, boxesT_ref,
                  keys_ref, meta_ref, tcls_ref,
                  loc_ref, cls_ref, objc_ref, noobj_ref, cnt_ref,
                  rows_sc, sem):
    b = pl.program_id(0)
    GG = G * G
    GG3 = 3 * GG

    @pl.when(b == 0)
    def _init():
        loc_ref[0] = 0.0
        cls_ref[0] = 0.0
        objc_ref[0] = 0.0
        noobj_ref[0] = 0.0
        cnt_ref[0] = 0.0

    for i in range(_N):
        key = keys_ref[0, 0, i]
        a = key // GG
        rem = key - a * GG
        ix = rem // G
        iy = rem - ix * G
        pltpu.make_async_copy(
            pred_hbm.at[b, a, ix, iy], rows_sc.at[i], sem).start()

    q = (jax.lax.broadcasted_iota(jnp.int32, (R, 128), 0) * 128
         + jax.lax.broadcasted_iota(jnp.int32, (R, 128), 1))
    a_idx = q // GG
    rem_q = q - a_idx * GG
    gx = (rem_q // G).astype(jnp.float32)
    gy = (rem_q - (rem_q // G) * G).astype(jnp.float32)
    valid = q < GG3
    aw0, aw1, aw2 = (float(_ANCHORS[3 * s + k, 0]) for k in range(3))
    ah0, ah1, ah2 = (float(_ANCHORS[3 * s + k, 1]) for k in range(3))
    awv = jnp.where(a_idx == 0, aw0, jnp.where(a_idx == 1, aw1, aw2))
    ahv = jnp.where(a_idx == 0, ah0, jnp.where(a_idx == 1, ah1, ah2))

    x0 = planes_ref[0, 0]
    x1 = planes_ref[0, 1]
    x2 = planes_ref[0, 2]
    x3 = planes_ref[0, 3]
    x4 = planes_ref[0, 4]
    px = (jax.nn.sigmoid(x0) + gx) * (1.0 / G)
    py = (jax.nn.sigmoid(x1) + gy) * (1.0 / G)
    pw = jnp.exp(x2) * awv
    ph = jnp.exp(x3) * ahv
    p1x = px - pw * 0.5
    p2x = px + pw * 0.5
    p1y = py - ph * 0.5
    p2y = py + ph * 0.5
    area_p = pw * ph
    ign = jnp.zeros((R, 128), jnp.bool_)
    for n in range(_N):
        bcx = boxes_s_ref[0, 0, 4 * n + 0]
        bcy = boxes_s_ref[0, 0, 4 * n + 1]
        bw = boxes_s_ref[0, 0, 4 * n + 2]
        bh = boxes_s_ref[0, 0, 4 * n + 3]
        iw = jnp.maximum(
            jnp.minimum(p2x, bcx + bw * 0.5) - jnp.maximum(p1x, bcx - bw * 0.5), 0.0)
        ih = jnp.maximum(
            jnp.minimum(p2y, bcy + bh * 0.5) - jnp.maximum(p1y, bcy - bh * 0.5), 0.0)
        inter = iw * ih
        ign = ign | (3.0 * inter > area_p + bw * bh)
    noobj_acc = jnp.sum(jnp.where(ign | (~valid), 0.0, _softplus(x4)))

    for i in range(_N):
        pltpu.make_async_copy(
            pred_hbm.at[b, 0, 0, 0], rows_sc.at[i], sem).wait()

    meta = meta_ref[0]
    lf = meta[:, 0:1]
    af = meta[:, 1:2]
    ixf = meta[:, 2:3]
    iyf = meta[:, 3:4]
    sx = meta[:, 4:5]
    sy = meta[:, 5:6]
    sw = meta[:, 6:7]
    sh = meta[:, 7:8]
    rows = rows_sc[:, :]
    tcls = tcls_ref[0]
    lane = jax.lax.broadcasted_iota(jnp.int32, (1, _C), 1)

    sig = jax.nn.sigmoid(rows)
    ex = jnp.exp(rows)
    sp = _softplus(rows)
    aw0f, aw1f, aw2f = (float(_ANCHORS[3 * s + k, 0]) for k in range(3))
    ah0f, ah1f, ah2f = (float(_ANCHORS[3 * s + k, 1]) for k in range(3))
    awcol = jnp.where(af == 0.0, aw0f, jnp.where(af == 1.0, aw1f, aw2f))
    ahcol = jnp.where(af == 0.0, ah0f, jnp.where(af == 1.0, ah1f, ah2f))
    addv = jnp.where(lane == 0, ixf, iyf)
    anchv = jnp.where(lane == 2, awcol, ahcol)
    tp_xy = (sig + addv) * (1.0 / G)
    tp_wh = ex * anchv
    tgt = jnp.where(lane == 0, sx,
                    jnp.where(lane == 1, sy,
                              jnp.where(lane == 2, sw, sh)))
    m_xy = (lane <= 1).astype(jnp.float32)
    m_wh = ((lane == 2) | (lane == 3)).astype(jnp.float32)
    m_obj = (lane == 4).astype(jnp.float32)
    m_cls = (lane >= 5).astype(jnp.float32)
    dxy = tp_xy - tgt
    dwh = tp_wh - tgt
    loc_c = jnp.sum(lf * (dxy * dxy * m_xy + dwh * dwh * m_wh))
    cls_c = jnp.sum(lf * (sp - rows * tcls) * m_cls)
    po = jnp.sum(rows * m_obj, axis=1, keepdims=True)
    posp = jnp.sum(sp * m_obj, axis=1, keepdims=True)
    pxc = jnp.sum(tp_xy * (lane == 0), axis=1, keepdims=True)
    pyc = jnp.sum(tp_xy * (lane == 1), axis=1, keepdims=True)
    pwc = jnp.sum(tp_wh * (lane == 2), axis=1, keepdims=True)
    phc = jnp.sum(tp_wh * (lane == 3), axis=1, keepdims=True)

    bt = boxesT_ref[0]
    gcxr = bt[0:1, :]
    gcyr = bt[1:2, :]
    gwr = bt[2:3, :]
    ghr = bt[3:4, :]
    g1xr = gcxr - gwr * 0.5
    g2xr = gcxr + gwr * 0.5
    g1yr = gcyr - ghr * 0.5
    g2yr = gcyr + ghr * 0.5
    area_gr = gwr * ghr
    iw2 = jnp.maximum(
        jnp.minimum(pxc + pwc * 0.5, g2xr) - jnp.maximum(pxc - pwc * 0.5, g1xr), 0.0)
    ih2 = jnp.maximum(
        jnp.minimum(pyc + phc * 0.5, g2yr) - jnp.maximum(pyc - phc * 0.5, g1yr), 0.0)
    inter2 = iw2 * ih2
    iou2 = inter2 / (pwc * phc + area_gr - inter2 + _EPS)
    ignf = (jnp.max(iou2, axis=1, keepdims=True) > _IGNORE_THRESHOLD)
    objc_c = jnp.sum(lf * ((posp - po)
                           - jnp.where(ignf, 0.0, 0.5 * posp)))
    cnt_c = jnp.sum(lf)

    loc_ref[0] += loc_c
    cls_ref[0] += cls_c
    objc_ref[0] += objc_c
    noobj_ref[0] += noobj_acc
    cnt_ref[0] += cnt_c


def _run_prep(boxes, labels_i):
    B = boxes.shape[0]
    outs = pl.pallas_call(
        _prep_kernel,
        in_specs=[
            pl.BlockSpec((B, _N, 4), lambda: (0, 0, 0)),
            pl.BlockSpec((B, _N), lambda: (0, 0)),
        ],
        out_specs=[
            spec
            for _ in range(3)
            for spec in (
                pl.BlockSpec((B, 1, _N), lambda: (0, 0, 0)),
                pl.BlockSpec((B, _N, 16), lambda: (0, 0, 0)),
                pl.BlockSpec((B, _N, _C), lambda: (0, 0, 0)),
            )
        ],
        out_shape=[
            shape
            for _ in range(3)
            for shape in (
                jax.ShapeDtypeStruct((B, 1, _N), jnp.int32),
                jax.ShapeDtypeStruct((B, _N, 16), jnp.float32),
                jax.ShapeDtypeStruct((B, _N, _C), jnp.float32),
            )
        ],
    )(boxes, labels_i)
    return outs


def _run_scale(s, pred, planes, boxes_flat, boxesT, keys, meta, tcls):
    B = pred.shape[0]
    G = pred.shape[2]
    R = planes.shape[2]
    outs = pl.pallas_call(
        functools.partial(_scale_kernel, s, G, R),
        grid=(B,),
        in_specs=[
            pl.BlockSpec(memory_space=pl.ANY),
            pl.BlockSpec((1, 5, R, 128), lambda b: (b, 0, 0, 0)),
            pl.BlockSpec((1, 1, 4 * _N), lambda b: (b, 0, 0), memory_space=pltpu.SMEM),
            pl.BlockSpec((1, 4, _N), lambda b: (b, 0, 0)),
            pl.BlockSpec((1, 1, _N), lambda b: (b, 0, 0), memory_space=pltpu.SMEM),
            pl.BlockSpec((1, _N, 16), lambda b: (b, 0, 0)),
            pl.BlockSpec((1, _N, _C), lambda b: (b, 0, 0)),
        ],
        out_specs=[
            pl.BlockSpec(memory_space=pltpu.SMEM) for _ in range(5)
        ],
        out_shape=[jax.ShapeDtypeStruct((1,), jnp.float32) for _ in range(5)],
        scratch_shapes=[
            pltpu.VMEM((_N, _C), jnp.float32),
            pltpu.SemaphoreType.DMA,
        ],
        compiler_params=pltpu.CompilerParams(
            dimension_semantics=("arbitrary",),
            allow_input_fusion=[False, True, False, False, False, False, False]),
    )(pred, planes, boxes_flat, boxesT, keys, meta, tcls)
    return outs


def _make_planes(pred):
    B = pred.shape[0]
    G = pred.shape[2]
    GG3 = 3 * G * G
    p5 = pred.reshape(B, GG3, _C)[:, :, 0:5]
    p5 = jax.lax.optimization_barrier(p5)
    planes = jnp.moveaxis(p5, 2, 1)
    R = (GG3 + 127) // 128
    pad = R * 128 - GG3
    planes = jnp.pad(planes, ((0, 0), (0, 0), (0, pad)))
    return planes.reshape(B, 5, R, 128)


def kernel(pred_large, pred_medium, pred_small, boxes, labels):
    B = pred_large.shape[0]
    boxes_flat = boxes.reshape(B, 1, 4 * _N)
    boxesT = jnp.swapaxes(boxes, 1, 2)
    labels_i = labels.astype(jnp.int32)
    prep = _run_prep(boxes, labels_i)
    loc = jnp.float32(0.0)
    cls = jnp.float32(0.0)
    obj = jnp.float32(0.0)
    cnt = jnp.float32(0.0)
    for s, pred in enumerate([pred_large, pred_medium, pred_small]):
        planes = _make_planes(pred)
        keys, meta, tcls = prep[3 * s], prep[3 * s + 1], prep[3 * s + 2]
        o_loc, o_cls, o_objc, o_noobj, o_cnt = _run_scale(
            s, pred, planes, boxes_flat, boxesT, keys, meta, tcls)
        loc = loc + o_loc[0]
        cls = cls + o_cls[0]
        obj = obj + o_objc[0] + _LAMBDA_NOOBJ * o_noobj[0]
        cnt = cnt + o_cnt[0]
    denom = jnp.maximum(1.0, cnt)
    loc_loss = loc / denom
    cls_loss = cls / denom
    obj_loss = obj / B
    total_loss = _BOX_GAIN * loc_loss + _OBJ_GAIN * obj_loss + _CLS_GAIN * cls_loss
    return total_loss, loc_loss, obj_loss, cls_loss
